# baseline (speedup 1.0000x reference)
import functools

import jax
import jax.numpy as jnp
from jax import lax
from jax.experimental import pallas as pl
from jax.experimental.pallas import tpu as pltpu
from jax.experimental.pallas import tpu_sc as plsc


_CH = 128


def _make_sc_gather(L_pad, D, n_chunks_per_worker, NC, NS):
    NW = NC * NS
    b_per_w = L_pad // NW
    mesh = plsc.VectorSubcoreMesh(core_axis_name="c", subcore_axis_name="s")

    @functools.partial(
        pl.kernel,
        mesh=mesh,
        out_type=jax.ShapeDtypeStruct((L_pad, D), jnp.float32),
        scratch_types=[
            pltpu.VMEM((_CH,), jnp.int32),
            pltpu.VMEM((_CH, D), jnp.float32),
            pltpu.SemaphoreType.DMA,
        ],
    )
    def gather_k(idx_hbm, table_hbm, out_hbm, idx_v, rows_v, sem):
        wid = lax.axis_index("s") * NC + lax.axis_index("c")
        base_w = wid * b_per_w

        def body(it, carry):
            base = base_w + it * _CH
            pltpu.sync_copy(idx_hbm.at[pl.ds(base, _CH)], idx_v)
            pltpu.async_copy(table_hbm.at[idx_v], rows_v, sem).wait()
            pltpu.sync_copy(rows_v, out_hbm.at[pl.ds(base, _CH)])
            return carry

        lax.fori_loop(0, n_chunks_per_worker, body, 0)

    return gather_k


_BR = 2000


def _mlp2(x, wi, bi, wo, bo):
    xb = x.astype(jnp.bfloat16)
    h = lax.dot_general(xb, wi.astype(jnp.bfloat16), (((1,), (1,)), ((), ())),
                        preferred_element_type=jnp.float32) + bi
    h = h * jnp.tanh(jax.nn.softplus(h))
    o = lax.dot_general(h.astype(jnp.bfloat16), wo.astype(jnp.bfloat16),
                        (((1,), (1,)), ((), ())),
                        preferred_element_type=jnp.float32) + bo
    return x + o


def _make_mlp_body(n_edge_blocks, D):
    def body(x_ref, wie, bie, woe, boe, wil, bil, wol, bol, o_ref):
        pid = pl.program_id(0)

        @pl.when(pid < n_edge_blocks)
        def _():
            x = x_ref[...].reshape(_BR // 2, 2 * D)
            o = _mlp2(x, wie[...], bie[...], woe[...], boe[...])
            o_ref[...] = o.reshape(_BR, D)

        @pl.when(pid >= n_edge_blocks)
        def _():
            o_ref[...] = _mlp2(x_ref[...], wil[...], bil[...], wol[...],
                               bol[...])

    return body


def _mlp_call(gathered, L, n_edge_blocks, D,
              wie, bie, woe, boe, wil, bil, wol, bol):
    grid = (L // _BR,)
    full = lambda i: (0, 0)
    return pl.pallas_call(
        _make_mlp_body(n_edge_blocks, D),
        grid=grid,
        in_specs=[
            pl.BlockSpec((_BR, D), lambda i: (i, 0)),
            pl.BlockSpec((2 * D, 2 * D), full),
            pl.BlockSpec((1, 2 * D), full),
            pl.BlockSpec((2 * D, 2 * D), full),
            pl.BlockSpec((1, 2 * D), full),
            pl.BlockSpec((D, D), full),
            pl.BlockSpec((1, D), full),
            pl.BlockSpec((D, D), full),
            pl.BlockSpec((1, D), full),
        ],
        out_specs=pl.BlockSpec((_BR, D), lambda i: (i, 0)),
        out_shape=jax.ShapeDtypeStruct((L, D), jnp.float32),
    )(gathered, wie, bie.reshape(1, -1), woe, boe.reshape(1, -1),
      wil, bil.reshape(1, -1), wol, bol.reshape(1, -1))


def kernel(node_embeddings, atoms_edge, atoms_label,
           W_inner_edge, b_inner_edge, W_outer_edge, b_outer_edge,
           W_inner_label, b_inner_label, W_outer_label, b_outer_label):
    N, D = node_embeddings.shape
    E2 = atoms_edge.shape[0]
    NL = atoms_label.shape[0]
    L = E2 + NL

    info = plsc.get_sparse_core_info()
    NC, NS = info.num_cores, info.num_subcores
    NW = NC * NS
    align = NW * _CH
    n_chunks_total = -(-L // align)
    L_pad = n_chunks_total * align
    n_chunks_per_worker = L_pad // (NW * _CH)

    pad = L_pad - L
    idx_all = jnp.concatenate([
        atoms_edge, atoms_label,
        jnp.zeros((pad,), dtype=jnp.int32),
    ])

    gather_k = _make_sc_gather(L_pad, D, n_chunks_per_worker, NC, NS)
    gathered = gather_k(idx_all, node_embeddings)

    n_edge_blocks = E2 // _BR
    output_messages = _mlp_call(
        gathered, L, n_edge_blocks, D,
        W_inner_edge, b_inner_edge, W_outer_edge, b_outer_edge,
        W_inner_label, b_inner_label, W_outer_label, b_outer_label)
    output_indices = idx_all[:L]
    return (output_messages, output_indices)


# speedup vs baseline: 1.0454x; 1.0454x over previous
import functools

import jax
import jax.numpy as jnp
from jax import lax
from jax.experimental import pallas as pl
from jax.experimental.pallas import tpu as pltpu
from jax.experimental.pallas import tpu_sc as plsc


_CH = 128


def _make_sc_gather(L_pad, D, n_chunks_per_worker, NC, NS):
    NW = NC * NS
    b_per_w = L_pad // NW
    mesh = plsc.VectorSubcoreMesh(core_axis_name="c", subcore_axis_name="s")

    @functools.partial(
        pl.kernel,
        mesh=mesh,
        out_type=jax.ShapeDtypeStruct((L_pad, D), jnp.float32),
        scratch_types=[
            pltpu.VMEM((_CH,), jnp.int32),
            pltpu.VMEM((_CH, D), jnp.float32),
            pltpu.SemaphoreType.DMA,
        ],
    )
    def gather_k(idx_hbm, table_hbm, out_hbm, idx_v, rows_v, sem):
        wid = lax.axis_index("s") * NC + lax.axis_index("c")
        base_w = wid * b_per_w

        def body(it, carry):
            base = base_w + it * _CH
            pltpu.sync_copy(idx_hbm.at[pl.ds(base, _CH)], idx_v)
            pltpu.async_copy(table_hbm.at[idx_v], rows_v, sem).wait()
            pltpu.sync_copy(rows_v, out_hbm.at[pl.ds(base, _CH)])
            return carry

        lax.fori_loop(0, n_chunks_per_worker, body, 0)

    return gather_k


_BR = 2000


def _mish(h):
    e = jnp.exp(h)
    u = e * (e + 2.0)
    return jnp.where(h > 20.0, h, h * (u / (u + 2.0)))


def _mlp2(x, wi, bi, wo, bo):
    xb = x.astype(jnp.bfloat16)
    h = lax.dot_general(xb, wi.astype(jnp.bfloat16), (((1,), (1,)), ((), ())),
                        preferred_element_type=jnp.float32) + bi
    h = _mish(h)
    o = lax.dot_general(h.astype(jnp.bfloat16), wo.astype(jnp.bfloat16),
                        (((1,), (1,)), ((), ())),
                        preferred_element_type=jnp.float32) + bo
    return x + o


def _make_mlp_body(n_edge_blocks, D):
    def body(x_ref, wie, bie, woe, boe, wil, bil, wol, bol, o_ref):
        pid = pl.program_id(0)

        @pl.when(pid < n_edge_blocks)
        def _():
            x = x_ref[...].reshape(_BR // 2, 2 * D)
            o = _mlp2(x, wie[...], bie[...], woe[...], boe[...])
            o_ref[...] = o.reshape(_BR, D)

        @pl.when(pid >= n_edge_blocks)
        def _():
            o_ref[...] = _mlp2(x_ref[...], wil[...], bil[...], wol[...],
                               bol[...])

    return body


def _mlp_call(gathered, L, n_edge_blocks, D,
              wie, bie, woe, boe, wil, bil, wol, bol):
    grid = (L // _BR,)
    full = lambda i: (0, 0)
    return pl.pallas_call(
        _make_mlp_body(n_edge_blocks, D),
        grid=grid,
        in_specs=[
            pl.BlockSpec((_BR, D), lambda i: (i, 0)),
            pl.BlockSpec((2 * D, 2 * D), full),
            pl.BlockSpec((1, 2 * D), full),
            pl.BlockSpec((2 * D, 2 * D), full),
            pl.BlockSpec((1, 2 * D), full),
            pl.BlockSpec((D, D), full),
            pl.BlockSpec((1, D), full),
            pl.BlockSpec((D, D), full),
            pl.BlockSpec((1, D), full),
        ],
        out_specs=pl.BlockSpec((_BR, D), lambda i: (i, 0)),
        out_shape=jax.ShapeDtypeStruct((L, D), jnp.float32),
    )(gathered, wie, bie.reshape(1, -1), woe, boe.reshape(1, -1),
      wil, bil.reshape(1, -1), wol, bol.reshape(1, -1))


def kernel(node_embeddings, atoms_edge, atoms_label,
           W_inner_edge, b_inner_edge, W_outer_edge, b_outer_edge,
           W_inner_label, b_inner_label, W_outer_label, b_outer_label):
    N, D = node_embeddings.shape
    E2 = atoms_edge.shape[0]
    NL = atoms_label.shape[0]
    L = E2 + NL

    info = plsc.get_sparse_core_info()
    NC, NS = info.num_cores, info.num_subcores
    NW = NC * NS
    align = NW * _CH
    n_chunks_total = -(-L // align)
    L_pad = n_chunks_total * align
    n_chunks_per_worker = L_pad // (NW * _CH)

    pad = L_pad - L
    idx_all = jnp.concatenate([
        atoms_edge, atoms_label,
        jnp.zeros((pad,), dtype=jnp.int32),
    ])

    gather_k = _make_sc_gather(L_pad, D, n_chunks_per_worker, NC, NS)
    gathered = gather_k(idx_all, node_embeddings)

    n_edge_blocks = E2 // _BR
    output_messages = _mlp_call(
        gathered, L, n_edge_blocks, D,
        W_inner_edge, b_inner_edge, W_outer_edge, b_outer_edge,
        W_inner_label, b_inner_label, W_outer_label, b_outer_label)
    output_indices = idx_all[:L]
    return (output_messages, output_indices)


# speedup vs baseline: 1.3658x; 1.3065x over previous
import functools

import jax
import jax.numpy as jnp
from jax import lax
from jax.experimental import pallas as pl
from jax.experimental.pallas import tpu as pltpu
from jax.experimental.pallas import tpu_sc as plsc


_CH = 128
_BR = 2048


def _make_sc_gather(rows, D, base, NC, NS):
    NW = NC * NS
    b_per_w = rows // NW
    n_it = b_per_w // _CH
    mesh = plsc.VectorSubcoreMesh(core_axis_name="c", subcore_axis_name="s")

    @functools.partial(
        pl.kernel,
        mesh=mesh,
        out_type=jax.ShapeDtypeStruct((rows, D), jnp.float32),
        scratch_types=[
            pltpu.VMEM((_CH,), jnp.int32),
            pltpu.VMEM((_CH, D), jnp.float32),
            pltpu.SemaphoreType.DMA,
        ],
    )
    def gather_k(idx_hbm, table_hbm, out_hbm, idx_v, rows_v, sem):
        wid = lax.axis_index("s") * NC + lax.axis_index("c")
        base_w = wid * b_per_w

        def body(it, carry):
            off = base_w + it * _CH
            pltpu.sync_copy(idx_hbm.at[pl.ds(base + off, _CH)], idx_v)
            pltpu.async_copy(table_hbm.at[idx_v], rows_v, sem).wait()
            pltpu.sync_copy(rows_v, out_hbm.at[pl.ds(off, _CH)])
            return carry

        lax.fori_loop(0, n_it, body, 0)

    return gather_k


def _mish(h):
    e = jnp.exp(h)
    u = e * (e + 2.0)
    return jnp.where(h > 20.0, h, h * (u / (u + 2.0)))


def _mlp2(x, wi, bi, wo, bo):
    xb = x.astype(jnp.bfloat16)
    h = lax.dot_general(xb, wi.astype(jnp.bfloat16), (((1,), (1,)), ((), ())),
                        preferred_element_type=jnp.float32) + bi
    h = _mish(h)
    o = lax.dot_general(h.astype(jnp.bfloat16), wo.astype(jnp.bfloat16),
                        (((1,), (1,)), ((), ())),
                        preferred_element_type=jnp.float32) + bo
    return x + o


def _make_mlp_body(base_block, bb, edge_rows_bb, D):
    def body(buf_ref, x_ref, wie, bie, woe, boe, wil, bil, wol, bol, o_ref):
        del buf_ref
        gpid = base_block + pl.program_id(0)

        @pl.when(gpid < bb)
        def _():
            x = x_ref[...].reshape(_BR // 2, 2 * D)
            o = _mlp2(x, wie[...], bie[...], woe[...], boe[...])
            o_ref[...] = o.reshape(_BR, D)

        @pl.when(gpid == bb)
        def _():
            xe = x_ref[:edge_rows_bb].reshape(edge_rows_bb // 2, 2 * D)
            oe = _mlp2(xe, wie[...], bie[...], woe[...], boe[...])
            o_ref[:edge_rows_bb] = oe.reshape(edge_rows_bb, D)
            xl = x_ref[edge_rows_bb:]
            o_ref[edge_rows_bb:] = _mlp2(xl, wil[...], bil[...], wol[...],
                                         bol[...])

        @pl.when(gpid > bb)
        def _():
            o_ref[...] = _mlp2(x_ref[...], wil[...], bil[...], wol[...],
                               bol[...])

    return body


def _mlp_chunk_call(buf, gathered_c, base_block, n_blocks, bb, edge_rows_bb,
                    L, D, weights):
    wie, bie, woe, boe, wil, bil, wol, bol = weights
    full = lambda j: (0, 0)
    in_specs = [
        pl.BlockSpec((8, D), full),
        pl.BlockSpec((_BR, D), lambda j: (j, 0)),
        pl.BlockSpec((2 * D, 2 * D), full),
        pl.BlockSpec((1, 2 * D), full),
        pl.BlockSpec((2 * D, 2 * D), full),
        pl.BlockSpec((1, 2 * D), full),
        pl.BlockSpec((D, D), full),
        pl.BlockSpec((1, D), full),
        pl.BlockSpec((D, D), full),
        pl.BlockSpec((1, D), full),
    ]
    return pl.pallas_call(
        _make_mlp_body(base_block, bb, edge_rows_bb, D),
        grid=(n_blocks,),
        in_specs=in_specs,
        out_specs=pl.BlockSpec((_BR, D), lambda j, b=base_block: (b + j, 0)),
        out_shape=jax.ShapeDtypeStruct((L, D), jnp.float32),
        input_output_aliases={0: 0},
    )(buf, gathered_c, wie, bie.reshape(1, -1), woe, boe.reshape(1, -1),
      wil, bil.reshape(1, -1), wol, bol.reshape(1, -1))


def kernel(node_embeddings, atoms_edge, atoms_label,
           W_inner_edge, b_inner_edge, W_outer_edge, b_outer_edge,
           W_inner_label, b_inner_label, W_outer_label, b_outer_label):
    N, D = node_embeddings.shape
    E2 = atoms_edge.shape[0]
    NL = atoms_label.shape[0]
    L = E2 + NL

    info = plsc.get_sparse_core_info()
    NC, NS = info.num_cores, info.num_subcores
    NW = NC * NS
    align = NW * _CH
    n_rounds = -(-L // align)
    L_pad = n_rounds * align
    assert L_pad % _BR == 0

    pad = L_pad - L
    idx_all = jnp.concatenate([
        atoms_edge, atoms_label,
        jnp.zeros((pad,), dtype=jnp.int32),
    ])

    K = 4
    per = (n_rounds // K) * align
    chunk_rows = [per] * (K - 1) + [L_pad - per * (K - 1)]
    bases = [per * c for c in range(K)]

    bb = E2 // _BR
    edge_rows_bb = E2 - bb * _BR
    weights = (W_inner_edge, b_inner_edge, W_outer_edge, b_outer_edge,
               W_inner_label, b_inner_label, W_outer_label, b_outer_label)

    gathered = []
    for c in range(K):
        gk = _make_sc_gather(chunk_rows[c], D, bases[c], NC, NS)
        gathered.append(gk(idx_all, node_embeddings))

    buf = None
    for c in range(K):
        base_block = bases[c] // _BR
        n_blocks = chunk_rows[c] // _BR
        if c == K - 1:
            n_blocks = -(-(L - bases[c]) // _BR)
        if buf is None:
            buf = pl.pallas_call(
                _make_mlp_body(base_block, bb, edge_rows_bb, D),
                grid=(n_blocks,),
                in_specs=[
                    pl.BlockSpec((8, D), lambda j: (0, 0)),
                    pl.BlockSpec((_BR, D), lambda j: (j, 0)),
                    pl.BlockSpec((2 * D, 2 * D), lambda j: (0, 0)),
                    pl.BlockSpec((1, 2 * D), lambda j: (0, 0)),
                    pl.BlockSpec((2 * D, 2 * D), lambda j: (0, 0)),
                    pl.BlockSpec((1, 2 * D), lambda j: (0, 0)),
                    pl.BlockSpec((D, D), lambda j: (0, 0)),
                    pl.BlockSpec((1, D), lambda j: (0, 0)),
                    pl.BlockSpec((D, D), lambda j: (0, 0)),
                    pl.BlockSpec((1, D), lambda j: (0, 0)),
                ],
                out_specs=pl.BlockSpec((_BR, D),
                                       lambda j, b=base_block: (b + j, 0)),
                out_shape=jax.ShapeDtypeStruct((L, D), jnp.float32),
            )(gathered[c][:8], gathered[c], weights[0],
              weights[1].reshape(1, -1), weights[2], weights[3].reshape(1, -1),
              weights[4], weights[5].reshape(1, -1), weights[6],
              weights[7].reshape(1, -1))
        else:
            buf = _mlp_chunk_call(buf, gathered[c], base_block, n_blocks, bb,
                                  edge_rows_bb, L, D, weights)

    output_indices = idx_all[:L]
    return (buf, output_indices)


# speedup vs baseline: 1.5151x; 1.1093x over previous
import functools

import jax
import jax.numpy as jnp
from jax import lax
from jax.experimental import pallas as pl
from jax.experimental.pallas import tpu as pltpu
from jax.experimental.pallas import tpu_sc as plsc


_CH = 128
_BR = 2048


def _make_sc_gather(rows, D, base, NC, NS):
    NW = NC * NS
    b_per_w = rows // NW
    n_it = b_per_w // _CH
    n_pairs = -(-n_it // 2)
    mesh = plsc.VectorSubcoreMesh(core_axis_name="c", subcore_axis_name="s")

    @functools.partial(
        pl.kernel,
        mesh=mesh,
        out_type=jax.ShapeDtypeStruct((rows, D), jnp.float32),
        scratch_types=[
            pltpu.VMEM((b_per_w,), jnp.int32),
            pltpu.VMEM((2, _CH, D), jnp.float32),
            pltpu.SemaphoreType.DMA,
            pltpu.SemaphoreType.DMA,
            pltpu.SemaphoreType.DMA,
            pltpu.SemaphoreType.DMA,
        ],
    )
    def gather_k(idx_hbm, table_hbm, out_hbm, idx_v, rows_v, g0, g1, w0, w1):
        wid = lax.axis_index("s") * NC + lax.axis_index("c")
        base_w = wid * b_per_w
        gsem = (g0, g1)
        wsem = (w0, w1)

        pltpu.sync_copy(idx_hbm.at[pl.ds(base + base_w, b_per_w)], idx_v)

        def start_gather(it, b):
            pltpu.async_copy(table_hbm.at[idx_v.at[pl.ds(it * _CH, _CH)]],
                             rows_v.at[b], gsem[b])

        def wait_gather(b):
            pltpu.make_async_copy(
                table_hbm.at[idx_v.at[pl.ds(0, _CH)]],
                rows_v.at[b], gsem[b]).wait()

        def start_writeout(it, b):
            pltpu.async_copy(rows_v.at[b],
                             out_hbm.at[pl.ds(base_w + it * _CH, _CH)],
                             wsem[b])

        def drain_writeout(b):
            pltpu.make_async_copy(rows_v.at[b],
                                  out_hbm.at[pl.ds(0, _CH)], wsem[b]).wait()

        def pair_body(p, carry):
            for b in range(2):
                it = 2 * p + b
                ob = 1 - b

                @pl.when(jnp.logical_and(it >= 2, it < n_it))
                def _():
                    drain_writeout(b)

                @pl.when(it < n_it)
                def _():
                    start_gather(it, b)

                @pl.when(it - 1 < n_it)
                def _():
                    @pl.when(it >= 1)
                    def _():
                        wait_gather(ob)
                        start_writeout(it - 1, ob)
            return carry

        lax.fori_loop(0, n_pairs, pair_body, 0)

        if n_it % 2 == 0:
            lb = (n_it - 1) % 2
            wait_gather(lb)
            start_writeout(n_it - 1, lb)
        drain_writeout(0)
        drain_writeout(1)

    return gather_k


def _mish(h):
    e = jnp.exp(h)
    u = e * (e + 2.0)
    return jnp.where(h > 20.0, h, h * (u / (u + 2.0)))


def _mlp2(x, wi, bi, wo, bo):
    xb = x.astype(jnp.bfloat16)
    h = lax.dot_general(xb, wi.astype(jnp.bfloat16), (((1,), (1,)), ((), ())),
                        preferred_element_type=jnp.float32) + bi
    h = _mish(h)
    o = lax.dot_general(h.astype(jnp.bfloat16), wo.astype(jnp.bfloat16),
                        (((1,), (1,)), ((), ())),
                        preferred_element_type=jnp.float32) + bo
    return x + o


def _make_mlp_body(base_block, bb, edge_rows_bb, D):
    def body(buf_ref, x_ref, wie, bie, woe, boe, wil, bil, wol, bol, o_ref):
        del buf_ref
        gpid = base_block + pl.program_id(0)

        @pl.when(gpid < bb)
        def _():
            x = x_ref[...].reshape(_BR // 2, 2 * D)
            o = _mlp2(x, wie[...], bie[...], woe[...], boe[...])
            o_ref[...] = o.reshape(_BR, D)

        @pl.when(gpid == bb)
        def _():
            xe = x_ref[:edge_rows_bb].reshape(edge_rows_bb // 2, 2 * D)
            oe = _mlp2(xe, wie[...], bie[...], woe[...], boe[...])
            o_ref[:edge_rows_bb] = oe.reshape(edge_rows_bb, D)
            xl = x_ref[edge_rows_bb:]
            o_ref[edge_rows_bb:] = _mlp2(xl, wil[...], bil[...], wol[...],
                                         bol[...])

        @pl.when(gpid > bb)
        def _():
            o_ref[...] = _mlp2(x_ref[...], wil[...], bil[...], wol[...],
                               bol[...])

    return body


def _mlp_chunk_call(buf, gathered_c, base_block, n_blocks, bb, edge_rows_bb,
                    L, D, weights):
    wie, bie, woe, boe, wil, bil, wol, bol = weights
    full = lambda j: (0, 0)
    in_specs = [
        pl.BlockSpec((8, D), full),
        pl.BlockSpec((_BR, D), lambda j: (j, 0)),
        pl.BlockSpec((2 * D, 2 * D), full),
        pl.BlockSpec((1, 2 * D), full),
        pl.BlockSpec((2 * D, 2 * D), full),
        pl.BlockSpec((1, 2 * D), full),
        pl.BlockSpec((D, D), full),
        pl.BlockSpec((1, D), full),
        pl.BlockSpec((D, D), full),
        pl.BlockSpec((1, D), full),
    ]
    return pl.pallas_call(
        _make_mlp_body(base_block, bb, edge_rows_bb, D),
        grid=(n_blocks,),
        in_specs=in_specs,
        out_specs=pl.BlockSpec((_BR, D), lambda j, b=base_block: (b + j, 0)),
        out_shape=jax.ShapeDtypeStruct((L, D), jnp.float32),
        input_output_aliases={0: 0},
    )(buf, gathered_c, wie, bie.reshape(1, -1), woe, boe.reshape(1, -1),
      wil, bil.reshape(1, -1), wol, bol.reshape(1, -1))


def kernel(node_embeddings, atoms_edge, atoms_label,
           W_inner_edge, b_inner_edge, W_outer_edge, b_outer_edge,
           W_inner_label, b_inner_label, W_outer_label, b_outer_label):
    N, D = node_embeddings.shape
    E2 = atoms_edge.shape[0]
    NL = atoms_label.shape[0]
    L = E2 + NL

    info = plsc.get_sparse_core_info()
    NC, NS = info.num_cores, info.num_subcores
    NW = NC * NS
    align = NW * _CH
    n_rounds = -(-L // align)
    L_pad = n_rounds * align
    assert L_pad % _BR == 0

    pad = L_pad - L
    idx_all = jnp.concatenate([
        atoms_edge, atoms_label,
        jnp.zeros((pad,), dtype=jnp.int32),
    ])

    K = 4
    per = (n_rounds // K) * align
    chunk_rows = [per] * (K - 1) + [L_pad - per * (K - 1)]
    bases = [per * c for c in range(K)]

    bb = E2 // _BR
    edge_rows_bb = E2 - bb * _BR
    weights = (W_inner_edge, b_inner_edge, W_outer_edge, b_outer_edge,
               W_inner_label, b_inner_label, W_outer_label, b_outer_label)

    gathered = []
    for c in range(K):
        gk = _make_sc_gather(chunk_rows[c], D, bases[c], NC, NS)
        gathered.append(gk(idx_all, node_embeddings))

    buf = None
    for c in range(K):
        base_block = bases[c] // _BR
        n_blocks = chunk_rows[c] // _BR
        if c == K - 1:
            n_blocks = -(-(L - bases[c]) // _BR)
        if buf is None:
            buf = pl.pallas_call(
                _make_mlp_body(base_block, bb, edge_rows_bb, D),
                grid=(n_blocks,),
                in_specs=[
                    pl.BlockSpec((8, D), lambda j: (0, 0)),
                    pl.BlockSpec((_BR, D), lambda j: (j, 0)),
                    pl.BlockSpec((2 * D, 2 * D), lambda j: (0, 0)),
                    pl.BlockSpec((1, 2 * D), lambda j: (0, 0)),
                    pl.BlockSpec((2 * D, 2 * D), lambda j: (0, 0)),
                    pl.BlockSpec((1, 2 * D), lambda j: (0, 0)),
                    pl.BlockSpec((D, D), lambda j: (0, 0)),
                    pl.BlockSpec((1, D), lambda j: (0, 0)),
                    pl.BlockSpec((D, D), lambda j: (0, 0)),
                    pl.BlockSpec((1, D), lambda j: (0, 0)),
                ],
                out_specs=pl.BlockSpec((_BR, D),
                                       lambda j, b=base_block: (b + j, 0)),
                out_shape=jax.ShapeDtypeStruct((L, D), jnp.float32),
            )(gathered[c][:8], gathered[c], weights[0],
              weights[1].reshape(1, -1), weights[2], weights[3].reshape(1, -1),
              weights[4], weights[5].reshape(1, -1), weights[6],
              weights[7].reshape(1, -1))
        else:
            buf = _mlp_chunk_call(buf, gathered[c], base_block, n_blocks, bb,
                                  edge_rows_bb, L, D, weights)

    output_indices = idx_all[:L]
    return (buf, output_indices)


# speedup vs baseline: 1.5174x; 1.0016x over previous
import functools

import jax
import jax.numpy as jnp
from jax import lax
from jax.experimental import pallas as pl
from jax.experimental.pallas import tpu as pltpu
from jax.experimental.pallas import tpu_sc as plsc


_CH = 128
_BR = 2048


def _make_sc_gather(rows, D, base, NC, NS):
    NW = NC * NS
    b_per_w = rows // NW
    n_it = b_per_w // _CH
    n_pairs = -(-n_it // 2)
    mesh = plsc.VectorSubcoreMesh(core_axis_name="c", subcore_axis_name="s")

    @functools.partial(
        pl.kernel,
        mesh=mesh,
        out_type=jax.ShapeDtypeStruct((rows, D), jnp.float32),
        scratch_types=[
            pltpu.VMEM((b_per_w,), jnp.int32),
            pltpu.VMEM((2, _CH, D), jnp.float32),
            pltpu.SemaphoreType.DMA,
            pltpu.SemaphoreType.DMA,
            pltpu.SemaphoreType.DMA,
            pltpu.SemaphoreType.DMA,
        ],
    )
    def gather_k(idx_hbm, table_hbm, out_hbm, idx_v, rows_v, g0, g1, w0, w1):
        wid = lax.axis_index("s") * NC + lax.axis_index("c")
        base_w = wid * b_per_w
        gsem = (g0, g1)
        wsem = (w0, w1)

        pltpu.sync_copy(idx_hbm.at[pl.ds(base + base_w, b_per_w)], idx_v)

        def start_gather(it, b):
            pltpu.async_copy(table_hbm.at[idx_v.at[pl.ds(it * _CH, _CH)]],
                             rows_v.at[b], gsem[b])

        def wait_gather(b):
            pltpu.make_async_copy(
                table_hbm.at[idx_v.at[pl.ds(0, _CH)]],
                rows_v.at[b], gsem[b]).wait()

        def start_writeout(it, b):
            pltpu.async_copy(rows_v.at[b],
                             out_hbm.at[pl.ds(base_w + it * _CH, _CH)],
                             wsem[b])

        def drain_writeout(b):
            pltpu.make_async_copy(rows_v.at[b],
                                  out_hbm.at[pl.ds(0, _CH)], wsem[b]).wait()

        def pair_body(p, carry):
            for b in range(2):
                it = 2 * p + b
                ob = 1 - b

                @pl.when(jnp.logical_and(it >= 2, it < n_it))
                def _():
                    drain_writeout(b)

                @pl.when(it < n_it)
                def _():
                    start_gather(it, b)

                @pl.when(it - 1 < n_it)
                def _():
                    @pl.when(it >= 1)
                    def _():
                        wait_gather(ob)
                        start_writeout(it - 1, ob)
            return carry

        lax.fori_loop(0, n_pairs, pair_body, 0)

        if n_it % 2 == 0:
            lb = (n_it - 1) % 2
            wait_gather(lb)
            start_writeout(n_it - 1, lb)
        drain_writeout(0)
        drain_writeout(1)

    return gather_k


def _mish(h):
    e = jnp.exp(h)
    u = e * (e + 2.0)
    return jnp.where(h > 20.0, h, h * (u / (u + 2.0)))


def _mlp2(x, wi, bi, wo, bo):
    xb = x.astype(jnp.bfloat16)
    h = lax.dot_general(xb, wi.astype(jnp.bfloat16), (((1,), (1,)), ((), ())),
                        preferred_element_type=jnp.float32) + bi
    h = _mish(h)
    o = lax.dot_general(h.astype(jnp.bfloat16), wo.astype(jnp.bfloat16),
                        (((1,), (1,)), ((), ())),
                        preferred_element_type=jnp.float32) + bo
    return x + o


def _make_mlp_body(base_block, bb, edge_rows_bb, D):
    def body(buf_ref, x_ref, wie, bie, woe, boe, wil, bil, wol, bol, o_ref):
        del buf_ref
        gpid = base_block + pl.program_id(0)

        @pl.when(gpid < bb)
        def _():
            x = x_ref[...].reshape(_BR // 2, 2 * D)
            o = _mlp2(x, wie[...], bie[...], woe[...], boe[...])
            o_ref[...] = o.reshape(_BR, D)

        @pl.when(gpid == bb)
        def _():
            xe = x_ref[:edge_rows_bb].reshape(edge_rows_bb // 2, 2 * D)
            oe = _mlp2(xe, wie[...], bie[...], woe[...], boe[...])
            o_ref[:edge_rows_bb] = oe.reshape(edge_rows_bb, D)
            xl = x_ref[edge_rows_bb:]
            o_ref[edge_rows_bb:] = _mlp2(xl, wil[...], bil[...], wol[...],
                                         bol[...])

        @pl.when(gpid > bb)
        def _():
            o_ref[...] = _mlp2(x_ref[...], wil[...], bil[...], wol[...],
                               bol[...])

    return body


def _mlp_chunk_call(buf, gathered_c, base_block, n_blocks, bb, edge_rows_bb,
                    L, D, weights):
    wie, bie, woe, boe, wil, bil, wol, bol = weights
    full = lambda j: (0, 0)
    in_specs = [
        pl.BlockSpec((8, D), full),
        pl.BlockSpec((_BR, D), lambda j: (j, 0)),
        pl.BlockSpec((2 * D, 2 * D), full),
        pl.BlockSpec((1, 2 * D), full),
        pl.BlockSpec((2 * D, 2 * D), full),
        pl.BlockSpec((1, 2 * D), full),
        pl.BlockSpec((D, D), full),
        pl.BlockSpec((1, D), full),
        pl.BlockSpec((D, D), full),
        pl.BlockSpec((1, D), full),
    ]
    return pl.pallas_call(
        _make_mlp_body(base_block, bb, edge_rows_bb, D),
        grid=(n_blocks,),
        in_specs=in_specs,
        out_specs=pl.BlockSpec((_BR, D), lambda j, b=base_block: (b + j, 0)),
        out_shape=jax.ShapeDtypeStruct((L, D), jnp.float32),
        input_output_aliases={0: 0},
    )(buf, gathered_c, wie, bie.reshape(1, -1), woe, boe.reshape(1, -1),
      wil, bil.reshape(1, -1), wol, bol.reshape(1, -1))


def kernel(node_embeddings, atoms_edge, atoms_label,
           W_inner_edge, b_inner_edge, W_outer_edge, b_outer_edge,
           W_inner_label, b_inner_label, W_outer_label, b_outer_label):
    N, D = node_embeddings.shape
    E2 = atoms_edge.shape[0]
    NL = atoms_label.shape[0]
    L = E2 + NL

    info = plsc.get_sparse_core_info()
    NC, NS = info.num_cores, info.num_subcores
    NW = NC * NS
    align = NW * _CH
    n_rounds = -(-L // align)
    L_pad = n_rounds * align
    assert L_pad % _BR == 0

    pad = L_pad - L
    idx_all = jnp.concatenate([
        atoms_edge, atoms_label,
        jnp.zeros((pad,), dtype=jnp.int32),
    ])

    first = min(8, n_rounds)
    rest = n_rounds - first
    K_rest = 4
    q, r = divmod(rest, K_rest)
    rounds = [first] + [q + 1] * r + [q] * (K_rest - r)
    rounds = [x for x in rounds if x > 0]
    K = len(rounds)
    chunk_rows = [x * align for x in rounds]
    bases = [sum(chunk_rows[:c]) for c in range(K)]

    bb = E2 // _BR
    edge_rows_bb = E2 - bb * _BR
    weights = (W_inner_edge, b_inner_edge, W_outer_edge, b_outer_edge,
               W_inner_label, b_inner_label, W_outer_label, b_outer_label)

    gathered = []
    for c in range(K):
        gk = _make_sc_gather(chunk_rows[c], D, bases[c], NC, NS)
        gathered.append(gk(idx_all, node_embeddings))

    buf = None
    for c in range(K):
        base_block = bases[c] // _BR
        n_blocks = chunk_rows[c] // _BR
        if c == K - 1:
            n_blocks = -(-(L - bases[c]) // _BR)
        if buf is None:
            buf = pl.pallas_call(
                _make_mlp_body(base_block, bb, edge_rows_bb, D),
                grid=(n_blocks,),
                in_specs=[
                    pl.BlockSpec((8, D), lambda j: (0, 0)),
                    pl.BlockSpec((_BR, D), lambda j: (j, 0)),
                    pl.BlockSpec((2 * D, 2 * D), lambda j: (0, 0)),
                    pl.BlockSpec((1, 2 * D), lambda j: (0, 0)),
                    pl.BlockSpec((2 * D, 2 * D), lambda j: (0, 0)),
                    pl.BlockSpec((1, 2 * D), lambda j: (0, 0)),
                    pl.BlockSpec((D, D), lambda j: (0, 0)),
                    pl.BlockSpec((1, D), lambda j: (0, 0)),
                    pl.BlockSpec((D, D), lambda j: (0, 0)),
                    pl.BlockSpec((1, D), lambda j: (0, 0)),
                ],
                out_specs=pl.BlockSpec((_BR, D),
                                       lambda j, b=base_block: (b + j, 0)),
                out_shape=jax.ShapeDtypeStruct((L, D), jnp.float32),
            )(gathered[c][:8], gathered[c], weights[0],
              weights[1].reshape(1, -1), weights[2], weights[3].reshape(1, -1),
              weights[4], weights[5].reshape(1, -1), weights[6],
              weights[7].reshape(1, -1))
        else:
            buf = _mlp_chunk_call(buf, gathered[c], base_block, n_blocks, bb,
                                  edge_rows_bb, L, D, weights)

    output_indices = idx_all[:L]
    return (buf, output_indices)
